# baseline (speedup 1.0000x reference)
import functools

import jax
import jax.numpy as jnp
from jax.experimental import pallas as pl
from jax.experimental.pallas import tpu as pltpu

B = 128
BEAM = 4
VOCAB = 100000
ROWS = 8
NEG = -3.0e38


def _row_kernel(x_ref, bs_ref, sc_ref, tok_ref):
    x = x_ref[...]
    m = jnp.max(x, axis=1, keepdims=True)
    sh = x - m
    s = jnp.sum(jnp.exp(sh), axis=1, keepdims=True)
    y = sh - jnp.log(s) + bs_ref[...]
    col = jax.lax.broadcasted_iota(jnp.int32, y.shape, 1)
    scs, toks = [], []
    for _ in range(4):
        v = jnp.max(y, axis=1, keepdims=True)
        idx = jnp.min(jnp.where(y == v, col, VOCAB), axis=1, keepdims=True)
        scs.append(v)
        toks.append(idx)
        y = jnp.where(col == idx, NEG, y)
    sc_ref[...] = jnp.concatenate(scs, axis=1)
    tok_ref[...] = jnp.concatenate(toks, axis=1)


def _merge_kernel(s_ref, t_ref, os_ref, ot_ref, ob_ref):
    s = s_ref[...]
    t = t_ref[...]
    slot = jax.lax.broadcasted_iota(jnp.int32, s.shape, 1)
    ss, tt, bb = [], [], []
    y = s
    for _ in range(4):
        v = jnp.max(y, axis=1, keepdims=True)
        sl = jnp.min(jnp.where(y == v, slot, 16), axis=1, keepdims=True)
        tok = jnp.max(jnp.where(slot == sl, t, -1), axis=1, keepdims=True)
        ss.append(v)
        tt.append(tok)
        bb.append(sl // 4)
        y = jnp.where(slot == sl, NEG, y)
    os_ref[...] = jnp.concatenate(ss, axis=1)
    ot_ref[...] = jnp.concatenate(tt, axis=1)
    ob_ref[...] = jnp.concatenate(bb, axis=1)


@jax.jit
def kernel(logits, beam_scores):
    b, beam, vocab = logits.shape
    rows = b * beam
    x = logits.reshape(rows, vocab)
    bs = beam_scores.reshape(rows, 1)

    sc, tok = pl.pallas_call(
        _row_kernel,
        grid=(rows // ROWS,),
        in_specs=[
            pl.BlockSpec((ROWS, vocab), lambda i: (i, 0)),
            pl.BlockSpec((ROWS, 1), lambda i: (i, 0)),
        ],
        out_specs=[
            pl.BlockSpec((ROWS, 4), lambda i: (i, 0)),
            pl.BlockSpec((ROWS, 4), lambda i: (i, 0)),
        ],
        out_shape=[
            jax.ShapeDtypeStruct((rows, 4), jnp.float32),
            jax.ShapeDtypeStruct((rows, 4), jnp.int32),
        ],
    )(x, bs)

    os_, ot, ob = pl.pallas_call(
        _merge_kernel,
        out_shape=[
            jax.ShapeDtypeStruct((b, 4), jnp.float32),
            jax.ShapeDtypeStruct((b, 4), jnp.int32),
            jax.ShapeDtypeStruct((b, 4), jnp.int32),
        ],
    )(sc.reshape(b, beam * 4), tok.reshape(b, beam * 4))

    return os_, ot, ob
